# Initial kernel scaffold; baseline (speedup 1.0000x reference)
import jax
import jax.numpy as jnp
from jax.experimental import pallas as pl


def kernel(x, edge_index, edge_pairs, W1, b1, W2, b2):
    raise NotImplementedError("write your pallas kernel here")


# speedup vs baseline: 3.6487x; 3.6487x over previous
import functools

import jax
import jax.numpy as jnp
from jax import lax
from jax.experimental import pallas as pl
from jax.experimental.pallas import tpu as pltpu
from jax.experimental.pallas import tpu_sc as plsc

N = 10000
E = 320000
D = 128

_TC_BLK = 1000


def _tc_body(x_ref, w1_ref, b1_ref, a_ref, b_ref):
    xv = x_ref[...]
    w1 = w1_ref[...]
    a_ref[...] = (
        jnp.dot(xv, w1[:D, :], preferred_element_type=jnp.float32) + b1_ref[...]
    )
    b_ref[...] = jnp.dot(xv, w1[D:, :], preferred_element_type=jnp.float32)


def _precompute_tables(x, W1, b1):
    return pl.pallas_call(
        _tc_body,
        grid=(N // _TC_BLK,),
        in_specs=[
            pl.BlockSpec((_TC_BLK, D), lambda i: (i, 0)),
            pl.BlockSpec((2 * D, D), lambda i: (0, 0)),
            pl.BlockSpec((1, D), lambda i: (0, 0)),
        ],
        out_specs=[
            pl.BlockSpec((_TC_BLK, D), lambda i: (i, 0)),
            pl.BlockSpec((_TC_BLK, D), lambda i: (i, 0)),
        ],
        out_shape=[
            jax.ShapeDtypeStruct((N, D), jnp.float32),
            jax.ShapeDtypeStruct((N, D), jnp.float32),
        ],
    )(x, W1, b1.reshape(1, D))


_INFO = plsc.get_sparse_core_info()
_NC = _INFO.num_cores
_NS = _INFO.num_subcores
_NW = _NC * _NS
_EPW = E // _NW
_BLK = 80
_NBLK = _EPW // _BLK

_mesh = plsc.VectorSubcoreMesh(core_axis_name="c", subcore_axis_name="s")


@functools.partial(
    pl.kernel,
    out_type=jax.ShapeDtypeStruct((E,), jnp.float32),
    mesh=_mesh,
    compiler_params=pltpu.CompilerParams(needs_layout_passes=False),
    scratch_types=[
        pltpu.VMEM((_BLK,), jnp.int32),
        pltpu.VMEM((_BLK,), jnp.int32),
        pltpu.VMEM((_BLK, D), jnp.float32),
        pltpu.VMEM((_BLK, D), jnp.float32),
        pltpu.VMEM((D,), jnp.float32),
        pltpu.VMEM((16,), jnp.float32),
        pltpu.VMEM((_BLK,), jnp.float32),
        pltpu.SemaphoreType.DMA,
    ],
)
def _sc_edges(a_hbm, b_hbm, idx_i_hbm, idx_j_hbm, w2_hbm, b2_hbm, out_hbm,
              idx_i_v, idx_j_v, rows_a, rows_b, w2_v, b2_v, logit_v, sem):
    wid = lax.axis_index("s") * _NC + lax.axis_index("c")
    pltpu.sync_copy(w2_hbm, w2_v)
    pltpu.sync_copy(b2_hbm, b2_v)
    base0 = wid * _EPW

    def blk_body(blk, carry):
        base = base0 + blk * _BLK
        pltpu.sync_copy(idx_i_hbm.at[pl.ds(base, _BLK)], idx_i_v)
        pltpu.sync_copy(idx_j_hbm.at[pl.ds(base, _BLK)], idx_j_v)
        cp_a = pltpu.async_copy(a_hbm.at[idx_i_v], rows_a, sem)
        cp_b = pltpu.async_copy(b_hbm.at[idx_j_v], rows_b, sem)
        cp_a.wait()
        cp_b.wait()

        bb = b2_v[...]
        lanes = lax.iota(jnp.int32, 16)

        def g_body(g, c):
            e0 = g * 16
            merged = jnp.zeros((16,), jnp.float32)
            for m in range(16):
                acc = jnp.zeros((16,), jnp.float32)
                for kk in range(D // 16):
                    a = rows_a[e0 + m, pl.ds(kk * 16, 16)]
                    b = rows_b[e0 + m, pl.ds(kk * 16, 16)]
                    h = jnp.maximum(a + b, 0.0)
                    acc = acc + h * w2_v[pl.ds(kk * 16, 16)]
                merged = jnp.where(lanes == m, jnp.sum(acc), merged)
            logit_v[pl.ds(e0, 16)] = 1.0 / (1.0 + jnp.exp(-(merged + bb)))
            return c

        lax.fori_loop(0, _BLK // 16, g_body, 0)
        pltpu.sync_copy(logit_v, out_hbm.at[pl.ds(base, _BLK)])
        return carry

    lax.fori_loop(0, _NBLK, blk_body, 0)


def kernel(x, edge_index, edge_pairs, W1, b1, W2, b2):
    del edge_index
    A, B = _precompute_tables(x, W1, b1)
    idx_i = edge_pairs[0].astype(jnp.int32)
    idx_j = edge_pairs[1].astype(jnp.int32)
    w2 = W2.reshape(D)
    b2b = jnp.broadcast_to(b2.reshape(1), (16,)).astype(jnp.float32)
    out = _sc_edges(A, B, idx_i, idx_j, w2, b2b)
    return out.reshape(E, 1)


# speedup vs baseline: 7.8988x; 2.1648x over previous
import functools

import jax
import jax.numpy as jnp
from jax import lax
from jax.experimental import pallas as pl
from jax.experimental.pallas import tpu as pltpu
from jax.experimental.pallas import tpu_sc as plsc

N = 10000
E = 320000
D = 128
DW = D // 2

_TC_BLK = 1000


def _tc_body(x_ref, w1_ref, b1_ref, a_ref, b_ref):
    xv = x_ref[...]
    w1 = w1_ref[...]
    a = jnp.dot(xv, w1[:D, :], preferred_element_type=jnp.float32) + b1_ref[...]
    b = jnp.dot(xv, w1[D:, :], preferred_element_type=jnp.float32)
    a_ref[...] = a.astype(jnp.bfloat16)
    b_ref[...] = b.astype(jnp.bfloat16)


def _precompute_tables(x, W1, b1):
    return pl.pallas_call(
        _tc_body,
        grid=(N // _TC_BLK,),
        in_specs=[
            pl.BlockSpec((_TC_BLK, D), lambda i: (i, 0)),
            pl.BlockSpec((2 * D, D), lambda i: (0, 0)),
            pl.BlockSpec((1, D), lambda i: (0, 0)),
        ],
        out_specs=[
            pl.BlockSpec((_TC_BLK, D), lambda i: (i, 0)),
            pl.BlockSpec((_TC_BLK, D), lambda i: (i, 0)),
        ],
        out_shape=[
            jax.ShapeDtypeStruct((N, D), jnp.bfloat16),
            jax.ShapeDtypeStruct((N, D), jnp.bfloat16),
        ],
    )(x, W1, b1.reshape(1, D))


_INFO = plsc.get_sparse_core_info()
_NC = _INFO.num_cores
_NS = _INFO.num_subcores
_NW = _NC * _NS
_EPW = E // _NW
_BLK = 80
_NBLK = _EPW // _BLK

_mesh = plsc.VectorSubcoreMesh(core_axis_name="c", subcore_axis_name="s")
_BF = jnp.bfloat16


@functools.partial(
    pl.kernel,
    out_type=jax.ShapeDtypeStruct((E,), jnp.float32),
    mesh=_mesh,
    compiler_params=pltpu.CompilerParams(
        needs_layout_passes=False, use_tc_tiling_on_sc=False),
    scratch_types=[
        pltpu.VMEM((_EPW,), jnp.int32),
        pltpu.VMEM((_EPW,), jnp.int32),
        pltpu.VMEM((2, _BLK, DW), jnp.int32),
        pltpu.VMEM((2, _BLK, DW), jnp.int32),
        pltpu.VMEM((DW,), jnp.int32),
        pltpu.VMEM((16,), jnp.float32),
        pltpu.VMEM((2, _BLK), jnp.float32),
        pltpu.SemaphoreType.DMA,
        pltpu.SemaphoreType.DMA,
        pltpu.SemaphoreType.DMA,
        pltpu.SemaphoreType.DMA,
        pltpu.SemaphoreType.DMA,
        pltpu.SemaphoreType.DMA,
    ],
)
def _sc_edges(a_hbm, b_hbm, idx_i_hbm, idx_j_hbm, w2_hbm, b2_hbm, out_hbm,
              idx_i_v, idx_j_v, rows_a, rows_b, w2_v, b2_v, logit_v,
              sa0, sa1, sb0, sb1, so0, so1):
    wid = lax.axis_index("s") * _NC + lax.axis_index("c")
    base0 = wid * _EPW
    pltpu.sync_copy(idx_i_hbm.at[pl.ds(base0, _EPW)], idx_i_v)
    pltpu.sync_copy(idx_j_hbm.at[pl.ds(base0, _EPW)], idx_j_v)
    pltpu.sync_copy(w2_hbm, w2_v)
    pltpu.sync_copy(b2_hbm, b2_v)

    sems_a = (sa0, sa1)
    sems_b = (sb0, sb1)
    sems_o = (so0, so1)

    def gather_desc(blk, slot):
        sl = pl.ds(blk * _BLK, _BLK)
        da = pltpu.make_async_copy(a_hbm.at[idx_i_v.at[sl]], rows_a.at[slot],
                                   sems_a[slot])
        db = pltpu.make_async_copy(b_hbm.at[idx_j_v.at[sl]], rows_b.at[slot],
                                   sems_b[slot])
        return da, db

    def out_desc(blk, slot):
        return pltpu.make_async_copy(
            logit_v.at[slot], out_hbm.at[pl.ds(base0 + blk * _BLK, _BLK)],
            sems_o[slot])

    def issue_gather(blk, slot):
        da, db = gather_desc(blk, slot)
        da.start()
        db.start()

    bb = b2_v[...]
    lanes = lax.iota(jnp.int32, 16)
    w2b = [plsc.bitcast(w2_v[pl.ds(kk * 16, 16)], _BF) for kk in range(DW // 16)]

    def compute(blk, slot):
        def g_body(g, c):
            e0 = g * 16
            merged = jnp.zeros((16,), jnp.float32)
            for m in range(16):
                acc = jnp.zeros((16,), jnp.float32)
                for kk in range(DW // 16):
                    a = plsc.bitcast(rows_a[slot, e0 + m, pl.ds(kk * 16, 16)], _BF)
                    b = plsc.bitcast(rows_b[slot, e0 + m, pl.ds(kk * 16, 16)], _BF)
                    h = jnp.maximum(a + b, jnp.zeros((32,), _BF))
                    p = h * w2b[kk]
                    p0, p1 = plsc.unpack(p, format=plsc.PackFormat.INTERLEAVED)
                    acc = acc + p0 + p1
                merged = jnp.where(lanes == m, jnp.sum(acc), merged)
            logit_v[slot, pl.ds(e0, 16)] = 1.0 / (1.0 + jnp.exp(-(merged + bb)))
            return c

        lax.fori_loop(0, _BLK // 16, g_body, 0)

    def process(blk, slot, first_round):
        da, db = gather_desc(blk, slot)
        da.wait()
        db.wait()
        if not first_round:
            out_desc(blk - 2, slot).wait()
        compute(blk, slot)
        out_desc(blk, slot).start()

        more = blk + 2 < _NBLK
        if isinstance(more, bool):
            if more:
                issue_gather(blk + 2, slot)
        else:
            @pl.when(more)
            def _():
                issue_gather(blk + 2, slot)

    issue_gather(0, 0)
    issue_gather(1, 1)
    process(0, 0, True)
    process(1, 1, True)

    def pair_body(g, c):
        blk = g * 2
        process(blk, 0, False)
        process(blk + 1, 1, False)
        return c

    lax.fori_loop(1, _NBLK // 2, pair_body, 0)
    process(_NBLK - 1, 0, False)
    out_desc(_NBLK - 1, 0).wait()
    out_desc(_NBLK - 2, 1).wait()


def kernel(x, edge_index, edge_pairs, W1, b1, W2, b2):
    del edge_index
    A16, B16 = _precompute_tables(x, W1, b1)
    a_pk = lax.bitcast_convert_type(A16.reshape(N, DW, 2), jnp.int32)
    b_pk = lax.bitcast_convert_type(B16.reshape(N, DW, 2), jnp.int32)
    idx_i = edge_pairs[0].astype(jnp.int32)
    idx_j = edge_pairs[1].astype(jnp.int32)
    w2_pk = lax.bitcast_convert_type(
        W2.reshape(D).astype(jnp.bfloat16).reshape(DW, 2), jnp.int32)
    b2b = jnp.broadcast_to(b2.reshape(1), (16,)).astype(jnp.float32)
    out = _sc_edges(a_pk, b_pk, idx_i, idx_j, w2_pk, b2b)
    return out.reshape(E, 1)


# speedup vs baseline: 8.7685x; 1.1101x over previous
import functools

import jax
import jax.numpy as jnp
from jax import lax
from jax.experimental import pallas as pl
from jax.experimental.pallas import tpu as pltpu
from jax.experimental.pallas import tpu_sc as plsc

N = 10000
E = 320000
D = 128
DW = D // 2

_TC_BLK = 1000


def _tc_body(x_ref, w1_ref, b1_ref, a_ref, b_ref):
    xv = x_ref[...]
    w1 = w1_ref[...]
    a = jnp.dot(xv, w1[:D, :], preferred_element_type=jnp.float32) + b1_ref[...]
    b = jnp.dot(xv, w1[D:, :], preferred_element_type=jnp.float32)
    a_ref[...] = a.astype(jnp.bfloat16)
    b_ref[...] = b.astype(jnp.bfloat16)


def _precompute_tables(x, W1, b1):
    return pl.pallas_call(
        _tc_body,
        grid=(N // _TC_BLK,),
        in_specs=[
            pl.BlockSpec((_TC_BLK, D), lambda i: (i, 0)),
            pl.BlockSpec((2 * D, D), lambda i: (0, 0)),
            pl.BlockSpec((1, D), lambda i: (0, 0)),
        ],
        out_specs=[
            pl.BlockSpec((_TC_BLK, D), lambda i: (i, 0)),
            pl.BlockSpec((_TC_BLK, D), lambda i: (i, 0)),
        ],
        out_shape=[
            jax.ShapeDtypeStruct((N, D), jnp.bfloat16),
            jax.ShapeDtypeStruct((N, D), jnp.bfloat16),
        ],
    )(x, W1, b1.reshape(1, D))


_INFO = plsc.get_sparse_core_info()
_NC = _INFO.num_cores
_NS = _INFO.num_subcores
_NW = _NC * _NS
_EPW = E // _NW
_BLK = 200
_BLKP = 208
_CHUNKS = ((0, 120), (120, 80))
_NBLK = _EPW // _BLK

_mesh = plsc.VectorSubcoreMesh(core_axis_name="c", subcore_axis_name="s")
_BF = jnp.bfloat16


@functools.partial(
    pl.kernel,
    out_type=jax.ShapeDtypeStruct((E,), jnp.float32),
    mesh=_mesh,
    compiler_params=pltpu.CompilerParams(
        needs_layout_passes=False, use_tc_tiling_on_sc=False),
    scratch_types=[
        pltpu.VMEM((_EPW,), jnp.int32),
        pltpu.VMEM((_EPW,), jnp.int32),
        pltpu.VMEM((2, _BLKP, DW), jnp.int32),
        pltpu.VMEM((2, _BLKP, DW), jnp.int32),
        pltpu.VMEM((DW,), jnp.int32),
        pltpu.VMEM((16,), jnp.float32),
        pltpu.VMEM((2, _BLKP), jnp.float32),
        pltpu.SemaphoreType.DMA,
        pltpu.SemaphoreType.DMA,
        pltpu.SemaphoreType.DMA,
        pltpu.SemaphoreType.DMA,
        pltpu.SemaphoreType.DMA,
        pltpu.SemaphoreType.DMA,
    ],
)
def _sc_edges(a_hbm, b_hbm, idx_i_hbm, idx_j_hbm, w2_hbm, b2_hbm, out_hbm,
              idx_i_v, idx_j_v, rows_a, rows_b, w2_v, b2_v, logit_v,
              sa0, sa1, sb0, sb1, so0, so1):
    wid = lax.axis_index("s") * _NC + lax.axis_index("c")
    base0 = wid * _EPW
    pltpu.sync_copy(idx_i_hbm.at[pl.ds(base0, _EPW)], idx_i_v)
    pltpu.sync_copy(idx_j_hbm.at[pl.ds(base0, _EPW)], idx_j_v)
    pltpu.sync_copy(w2_hbm, w2_v)
    pltpu.sync_copy(b2_hbm, b2_v)

    sems_a = (sa0, sa1)
    sems_b = (sb0, sb1)
    sems_o = (so0, so1)

    def gather_descs(blk, slot):
        descs = []
        for off, ln in _CHUNKS:
            sl = pl.ds(blk * _BLK + off, ln)
            dst = pl.ds(off, ln)
            descs.append(pltpu.make_async_copy(
                a_hbm.at[idx_i_v.at[sl]], rows_a.at[slot].at[dst],
                sems_a[slot]))
            descs.append(pltpu.make_async_copy(
                b_hbm.at[idx_j_v.at[sl]], rows_b.at[slot].at[dst],
                sems_b[slot]))
        return descs

    def out_desc(blk, slot):
        return pltpu.make_async_copy(
            logit_v.at[slot].at[pl.ds(0, _BLK)],
            out_hbm.at[pl.ds(base0 + blk * _BLK, _BLK)],
            sems_o[slot])

    def issue_gather(blk, slot):
        for d in gather_descs(blk, slot):
            d.start()

    bb = b2_v[...]
    lanes = lax.iota(jnp.int32, 16)
    w2b = [plsc.bitcast(w2_v[pl.ds(kk * 16, 16)], _BF) for kk in range(DW // 16)]

    def compute(blk, slot):
        def g_body(g, c):
            e0 = g * 16
            merged = jnp.zeros((16,), jnp.float32)
            zero_b = jnp.zeros((32,), _BF)
            for m in range(16):
                accb = zero_b
                for kk in range(DW // 16):
                    a = plsc.bitcast(rows_a[slot, e0 + m, pl.ds(kk * 16, 16)], _BF)
                    b = plsc.bitcast(rows_b[slot, e0 + m, pl.ds(kk * 16, 16)], _BF)
                    h = jnp.maximum(a + b, zero_b)
                    accb = accb + h * w2b[kk]
                p0, p1 = plsc.unpack(accb, format=plsc.PackFormat.INTERLEAVED)
                merged = jnp.where(lanes == m, jnp.sum(p0 + p1), merged)
            logit_v[slot, pl.ds(e0, 16)] = 1.0 / (1.0 + jnp.exp(-(merged + bb)))
            return c

        lax.fori_loop(0, _BLKP // 16, g_body, 0)

    def process(blk, slot, first_round):
        for d in gather_descs(blk, slot):
            d.wait()
        if not first_round:
            out_desc(blk - 2, slot).wait()
        compute(blk, slot)
        out_desc(blk, slot).start()

        more = blk + 2 < _NBLK
        if isinstance(more, bool):
            if more:
                issue_gather(blk + 2, slot)
        else:
            @pl.when(more)
            def _():
                issue_gather(blk + 2, slot)

    issue_gather(0, 0)
    issue_gather(1, 1)
    process(0, 0, True)
    process(1, 1, True)

    def pair_body(g, c):
        blk = g * 2
        process(blk, 0, False)
        process(blk + 1, 1, False)
        return c

    lax.fori_loop(1, _NBLK // 2, pair_body, 0)
    out_desc(_NBLK - 2, 0).wait()
    out_desc(_NBLK - 1, 1).wait()


def kernel(x, edge_index, edge_pairs, W1, b1, W2, b2):
    del edge_index
    A16, B16 = _precompute_tables(x, W1, b1)
    a_pk = lax.bitcast_convert_type(A16.reshape(N, DW, 2), jnp.int32)
    b_pk = lax.bitcast_convert_type(B16.reshape(N, DW, 2), jnp.int32)
    idx_i = edge_pairs[0].astype(jnp.int32)
    idx_j = edge_pairs[1].astype(jnp.int32)
    w2_pk = lax.bitcast_convert_type(
        W2.reshape(D).astype(jnp.bfloat16).reshape(DW, 2), jnp.int32)
    b2b = jnp.broadcast_to(b2.reshape(1), (16,)).astype(jnp.float32)
    out = _sc_edges(a_pk, b_pk, idx_i, idx_j, w2_pk, b2b)
    return out.reshape(E, 1)
